# initial kernel scaffold (unmeasured)
import jax
import jax.numpy as jnp
from jax import lax
from jax.experimental import pallas as pl
from jax.experimental.pallas import tpu as pltpu

N_DEV = 8
M, N = 4096, 2048
CHUNK = M // N_DEV
N_HOPS = 2 * (N_DEV - 1)


def _ar_body(p_ref, out_ref, comm_ref, send_buf, send_sems, recv_sems):
    my = lax.axis_index("i")
    left = (my - 1) % N_DEV
    right = (my + 1) % N_DEV

    barrier_sem = pltpu.get_barrier_semaphore()
    for nbr in (left, right):
        pl.semaphore_signal(
            barrier_sem, inc=1,
            device_id=(nbr,), device_id_type=pl.DeviceIdType.MESH,
        )
    pl.semaphore_wait(barrier_sem, 2)

    def chunk_rows(c):
        return pl.ds(c * CHUNK, CHUNK)

    send_buf[...] = p_ref[chunk_rows(my), :]
    for s in range(N_DEV - 1):
        rdma = pltpu.make_async_remote_copy(
            src_ref=send_buf,
            dst_ref=comm_ref.at[s],
            send_sem=send_sems.at[s],
            recv_sem=recv_sems.at[s],
            device_id=(right,),
            device_id_type=pl.DeviceIdType.MESH,
        )
        rdma.start()
        rdma.wait()

        c = (my - s - 1) % N_DEV
        acc = (comm_ref[s].astype(jnp.float32)
               + p_ref[chunk_rows(c), :].astype(jnp.float32))
        send_buf[...] = acc.astype(jnp.bfloat16)
    owned = (my + 1) % N_DEV
    out_ref[chunk_rows(owned), :] = send_buf[...].astype(jnp.float32)

    for g in range(N_DEV - 1):
        slot = (N_DEV - 1) + g
        rdma = pltpu.make_async_remote_copy(
            src_ref=send_buf,
            dst_ref=comm_ref.at[slot],
            send_sem=send_sems.at[slot],
            recv_sem=recv_sems.at[slot],
            device_id=(right,),
            device_id_type=pl.DeviceIdType.MESH,
        )
        rdma.start()
        rdma.wait()

        c = (my - g) % N_DEV
        out_ref[chunk_rows(c), :] = comm_ref[slot].astype(jnp.float32)
        if g < N_DEV - 2:
            send_buf[...] = comm_ref[slot]


def _ring_all_reduce(partial):
    return pl.pallas_call(
        _ar_body,
        out_shape=jax.ShapeDtypeStruct((M, N), jnp.float32),
        in_specs=[pl.BlockSpec(memory_space=pltpu.VMEM)],
        out_specs=pl.BlockSpec(memory_space=pltpu.VMEM),
        scratch_shapes=[
            pltpu.VMEM((N_HOPS, CHUNK, N), jnp.bfloat16),
            pltpu.VMEM((CHUNK, N), jnp.bfloat16),
            pltpu.SemaphoreType.DMA((N_HOPS,)),
            pltpu.SemaphoreType.DMA((N_HOPS,)),
        ],
        compiler_params=pltpu.CompilerParams(
            collective_id=0,
            vmem_limit_bytes=128 * 1024 * 1024,
        ),
    )(partial)


def kernel(x, w_mat):
    partial = jnp.dot(x, w_mat, preferred_element_type=jnp.bfloat16)
    y = _ring_all_reduce(partial)
    amax = jnp.max(jnp.abs(y))
    scale = amax / 448.0
    q = (y / scale).astype(jnp.float8_e4m3fn)
    return q.astype(jnp.float32) * scale


# baseline (device time: 368396 ns/iter reference)
import jax
import jax.numpy as jnp
from jax import lax
from jax.experimental import pallas as pl
from jax.experimental.pallas import tpu as pltpu

N_DEV = 8
M, N = 4096, 2048
CHUNK = M // N_DEV


def _ar_body(p_ref, out_ref, a2a_ref, ag_ref, send_buf,
             a2a_send_sems, a2a_recv_sems, ag_send_sems, ag_recv_sems):
    my = lax.axis_index("i")
    left = (my - 1) % N_DEV
    right = (my + 1) % N_DEV

    def chunk_rows(c):
        return pl.ds(c * CHUNK, CHUNK)

    barrier_sem = pltpu.get_barrier_semaphore()
    for k in range(1, N_DEV):
        pl.semaphore_signal(
            barrier_sem, inc=1,
            device_id=((my + k) % N_DEV,),
            device_id_type=pl.DeviceIdType.MESH,
        )
    pl.semaphore_wait(barrier_sem, N_DEV - 1)

    rdmas = []
    for k in range(1, N_DEV):
        d = (my + k) % N_DEV
        slot = N_DEV - 1 - k
        rdma = pltpu.make_async_remote_copy(
            src_ref=p_ref.at[chunk_rows(d)],
            dst_ref=a2a_ref.at[slot],
            send_sem=a2a_send_sems.at[slot],
            recv_sem=a2a_recv_sems.at[slot],
            device_id=(d,),
            device_id_type=pl.DeviceIdType.MESH,
        )
        rdma.start()
        rdmas.append(rdma)
    for rdma in rdmas:
        rdma.wait()

    acc = p_ref[chunk_rows(my), :].astype(jnp.float32)
    for s in range(N_DEV - 1):
        acc = acc + a2a_ref[s].astype(jnp.float32)
    send_buf[...] = acc.astype(jnp.bfloat16)
    out_ref[chunk_rows(my), :] = send_buf[...]

    for g in range(N_DEV - 1):
        rdma = pltpu.make_async_remote_copy(
            src_ref=send_buf,
            dst_ref=ag_ref.at[g],
            send_sem=ag_send_sems.at[g],
            recv_sem=ag_recv_sems.at[g],
            device_id=(right,),
            device_id_type=pl.DeviceIdType.MESH,
        )
        rdma.start()
        rdma.wait()

        c = (my - g - 1) % N_DEV
        out_ref[chunk_rows(c), :] = ag_ref[g]
        if g < N_DEV - 2:
            send_buf[...] = ag_ref[g]


def _all_reduce(partial):
    return pl.pallas_call(
        _ar_body,
        out_shape=jax.ShapeDtypeStruct((M, N), jnp.bfloat16),
        in_specs=[pl.BlockSpec(memory_space=pltpu.VMEM)],
        out_specs=pl.BlockSpec(memory_space=pltpu.VMEM),
        scratch_shapes=[
            pltpu.VMEM((N_DEV - 1, CHUNK, N), jnp.bfloat16),
            pltpu.VMEM((N_DEV - 1, CHUNK, N), jnp.bfloat16),
            pltpu.VMEM((CHUNK, N), jnp.bfloat16),
            pltpu.SemaphoreType.DMA((N_DEV - 1,)),
            pltpu.SemaphoreType.DMA((N_DEV - 1,)),
            pltpu.SemaphoreType.DMA((N_DEV - 1,)),
            pltpu.SemaphoreType.DMA((N_DEV - 1,)),
        ],
        compiler_params=pltpu.CompilerParams(
            collective_id=0,
            vmem_limit_bytes=128 * 1024 * 1024,
        ),
    )(partial)


def kernel(x, w_mat):
    partial = jnp.dot(x, w_mat, preferred_element_type=jnp.bfloat16)
    y = _all_reduce(partial).astype(jnp.float32)
    amax = jnp.max(jnp.abs(y))
    scale = amax / 448.0
    q = (y / scale).astype(jnp.float8_e4m3fn)
    q = jax.lax.optimization_barrier(q)
    return q.astype(jnp.float32) * scale


# device time: 332627 ns/iter; 1.1075x vs baseline; 1.1075x over previous
import jax
import jax.numpy as jnp
from jax import lax
from jax.experimental import pallas as pl
from jax.experimental.pallas import tpu as pltpu

N_DEV = 8
M, N = 4096, 2048
CHUNK = M // N_DEV


def _ar_body(p_ref, out_ref, a2a_ref, ag_ref, send_buf,
             a2a_send_sems, a2a_recv_sems, ag_send_sems, ag_recv_sems):
    my = lax.axis_index("i")

    def chunk_rows(c):
        return pl.ds(c * CHUNK, CHUNK)

    barrier_sem = pltpu.get_barrier_semaphore()
    for k in range(1, N_DEV):
        pl.semaphore_signal(
            barrier_sem, inc=1,
            device_id=((my + k) % N_DEV,),
            device_id_type=pl.DeviceIdType.MESH,
        )
    pl.semaphore_wait(barrier_sem, N_DEV - 1)

    rdmas = []
    for k in range(1, N_DEV):
        d = (my + k) % N_DEV
        slot = N_DEV - 1 - k
        rdma = pltpu.make_async_remote_copy(
            src_ref=p_ref.at[chunk_rows(d)],
            dst_ref=a2a_ref.at[slot],
            send_sem=a2a_send_sems.at[slot],
            recv_sem=a2a_recv_sems.at[slot],
            device_id=(d,),
            device_id_type=pl.DeviceIdType.MESH,
        )
        rdma.start()
        rdmas.append(rdma)
    for rdma in rdmas:
        rdma.wait()

    acc = p_ref[chunk_rows(my), :].astype(jnp.float32)
    for s in range(N_DEV - 1):
        acc = acc + a2a_ref[s].astype(jnp.float32)
    send_buf[...] = acc.astype(jnp.bfloat16)
    out_ref[chunk_rows(my), :] = send_buf[...]

    ag_rdmas = []
    for k in range(1, N_DEV):
        d = (my + k) % N_DEV
        slot = N_DEV - 1 - k
        rdma = pltpu.make_async_remote_copy(
            src_ref=send_buf,
            dst_ref=ag_ref.at[slot],
            send_sem=ag_send_sems.at[slot],
            recv_sem=ag_recv_sems.at[slot],
            device_id=(d,),
            device_id_type=pl.DeviceIdType.MESH,
        )
        rdma.start()
        ag_rdmas.append(rdma)
    for k, rdma in enumerate(ag_rdmas, start=1):
        rdma.wait()
        c = (my - k) % N_DEV
        out_ref[chunk_rows(c), :] = ag_ref[N_DEV - 1 - k]


def _all_reduce(partial):
    return pl.pallas_call(
        _ar_body,
        out_shape=jax.ShapeDtypeStruct((M, N), jnp.bfloat16),
        in_specs=[pl.BlockSpec(memory_space=pltpu.VMEM)],
        out_specs=pl.BlockSpec(memory_space=pltpu.VMEM),
        scratch_shapes=[
            pltpu.VMEM((N_DEV - 1, CHUNK, N), jnp.bfloat16),
            pltpu.VMEM((N_DEV - 1, CHUNK, N), jnp.bfloat16),
            pltpu.VMEM((CHUNK, N), jnp.bfloat16),
            pltpu.SemaphoreType.DMA((N_DEV - 1,)),
            pltpu.SemaphoreType.DMA((N_DEV - 1,)),
            pltpu.SemaphoreType.DMA((N_DEV - 1,)),
            pltpu.SemaphoreType.DMA((N_DEV - 1,)),
        ],
        compiler_params=pltpu.CompilerParams(
            collective_id=0,
            vmem_limit_bytes=128 * 1024 * 1024,
        ),
    )(partial)


def kernel(x, w_mat):
    partial = jnp.dot(x, w_mat, preferred_element_type=jnp.bfloat16)
    y = _all_reduce(partial).astype(jnp.float32)
    amax = jnp.max(jnp.abs(y))
    scale = amax / 448.0
    q = (y / scale).astype(jnp.float8_e4m3fn)
    q = jax.lax.optimization_barrier(q)
    return q.astype(jnp.float32) * scale


# device time: 260345 ns/iter; 1.4150x vs baseline; 1.2776x over previous
import jax
import jax.numpy as jnp
from jax import lax
from jax.experimental import pallas as pl
from jax.experimental.pallas import tpu as pltpu

N_DEV = 8
M, N = 4096, 2048
CHUNK = M // N_DEV


def _ar_body(p_ref, out_ref, a2a_ref, agq_ref, q_send, amax_send, amax_ref,
             a2a_send_sems, a2a_recv_sems, ag_send_sems, ag_recv_sems,
             am_send_sems, am_recv_sems):
    my = lax.axis_index("i")

    def chunk_rows(c):
        return pl.ds(c * CHUNK, CHUNK)

    barrier_sem = pltpu.get_barrier_semaphore()
    for k in range(1, N_DEV):
        pl.semaphore_signal(
            barrier_sem, inc=1,
            device_id=((my + k) % N_DEV,),
            device_id_type=pl.DeviceIdType.MESH,
        )
    pl.semaphore_wait(barrier_sem, N_DEV - 1)

    rs_rdmas = []
    for k in range(1, N_DEV):
        d = (my + k) % N_DEV
        slot = N_DEV - 1 - k
        rdma = pltpu.make_async_remote_copy(
            src_ref=p_ref.at[chunk_rows(d)],
            dst_ref=a2a_ref.at[slot],
            send_sem=a2a_send_sems.at[slot],
            recv_sem=a2a_recv_sems.at[slot],
            device_id=(d,),
            device_id_type=pl.DeviceIdType.MESH,
        )
        rdma.start()
        rs_rdmas.append(rdma)
    for rdma in rs_rdmas:
        rdma.wait()

    acc = p_ref[chunk_rows(my), :].astype(jnp.float32)
    for s in range(N_DEV - 1):
        acc = acc + a2a_ref[s].astype(jnp.float32)
    my_amax = jnp.max(jnp.abs(acc))

    amax_send[...] = jnp.zeros((8, 128), jnp.float32) + my_amax
    am_rdmas = []
    for k in range(1, N_DEV):
        d = (my + k) % N_DEV
        slot = N_DEV - 1 - k
        rdma = pltpu.make_async_remote_copy(
            src_ref=amax_send,
            dst_ref=amax_ref.at[slot],
            send_sem=am_send_sems.at[slot],
            recv_sem=am_recv_sems.at[slot],
            device_id=(d,),
            device_id_type=pl.DeviceIdType.MESH,
        )
        rdma.start()
        am_rdmas.append(rdma)
    for rdma in am_rdmas:
        rdma.wait()
    amax = jnp.maximum(jnp.max(amax_ref[...]), my_amax)
    scale = amax / 448.0

    q = (acc / scale).astype(jnp.float8_e4m3fn)
    q_send[...] = q
    out_ref[chunk_rows(my), :] = (q.astype(jnp.float32) * scale).astype(jnp.bfloat16)

    ag_rdmas = []
    for k in range(1, N_DEV):
        d = (my + k) % N_DEV
        slot = N_DEV - 1 - k
        rdma = pltpu.make_async_remote_copy(
            src_ref=q_send,
            dst_ref=agq_ref.at[slot],
            send_sem=ag_send_sems.at[slot],
            recv_sem=ag_recv_sems.at[slot],
            device_id=(d,),
            device_id_type=pl.DeviceIdType.MESH,
        )
        rdma.start()
        ag_rdmas.append(rdma)
    for k, rdma in enumerate(ag_rdmas, start=1):
        rdma.wait()
        c = (my - k) % N_DEV
        out_ref[chunk_rows(c), :] = (
            agq_ref[N_DEV - 1 - k].astype(jnp.float32) * scale
        ).astype(jnp.bfloat16)


def _all_reduce_quant(partial):
    n_sl = N_DEV - 1
    return pl.pallas_call(
        _ar_body,
        out_shape=jax.ShapeDtypeStruct((M, N), jnp.bfloat16),
        in_specs=[pl.BlockSpec(memory_space=pltpu.VMEM)],
        out_specs=pl.BlockSpec(memory_space=pltpu.VMEM),
        scratch_shapes=[
            pltpu.VMEM((n_sl, CHUNK, N), jnp.bfloat16),
            pltpu.VMEM((n_sl, CHUNK, N), jnp.float8_e4m3fn),
            pltpu.VMEM((CHUNK, N), jnp.float8_e4m3fn),
            pltpu.VMEM((8, 128), jnp.float32),
            pltpu.VMEM((n_sl, 8, 128), jnp.float32),
            pltpu.SemaphoreType.DMA((n_sl,)),
            pltpu.SemaphoreType.DMA((n_sl,)),
            pltpu.SemaphoreType.DMA((n_sl,)),
            pltpu.SemaphoreType.DMA((n_sl,)),
            pltpu.SemaphoreType.DMA((n_sl,)),
            pltpu.SemaphoreType.DMA((n_sl,)),
        ],
        compiler_params=pltpu.CompilerParams(
            collective_id=0,
            vmem_limit_bytes=128 * 1024 * 1024,
        ),
    )(partial)


def kernel(x, w_mat):
    partial = jnp.dot(x, w_mat, preferred_element_type=jnp.bfloat16)
    return _all_reduce_quant(partial).astype(jnp.float32)


# device time: 253678 ns/iter; 1.4522x vs baseline; 1.0263x over previous
import jax
import jax.numpy as jnp
from jax import lax
from jax.experimental import pallas as pl
from jax.experimental.pallas import tpu as pltpu

N_DEV = 8
M, N = 4096, 2048
CHUNK = M // N_DEV


def _ar_body(p_ref, out_ref, a2a_ref, agq_ref, q_send, amax_send, amax_ref,
             a2a_send_sems, a2a_recv_sems, ag_send_sems, ag_recv_sems,
             am_send_sems, am_recv_sems):
    my = lax.axis_index("i")

    def chunk_rows(c):
        return pl.ds(c * CHUNK, CHUNK)

    barrier_sem = pltpu.get_barrier_semaphore()
    for k in range(1, N_DEV):
        pl.semaphore_signal(
            barrier_sem, inc=1,
            device_id=((my + k) % N_DEV,),
            device_id_type=pl.DeviceIdType.MESH,
        )
    pl.semaphore_wait(barrier_sem, N_DEV - 1)

    rs_rdmas = []
    for k in range(1, N_DEV):
        d = (my + k) % N_DEV
        slot = N_DEV - 1 - k
        rdma = pltpu.make_async_remote_copy(
            src_ref=p_ref.at[chunk_rows(d)],
            dst_ref=a2a_ref.at[slot],
            send_sem=a2a_send_sems.at[slot],
            recv_sem=a2a_recv_sems.at[slot],
            device_id=(d,),
            device_id_type=pl.DeviceIdType.MESH,
        )
        rdma.start()
        rs_rdmas.append(rdma)
    acc = p_ref[chunk_rows(my), :].astype(jnp.float32)
    for k, rdma in enumerate(rs_rdmas, start=1):
        rdma.wait()
        acc = acc + a2a_ref[N_DEV - 1 - k].astype(jnp.float32)
    my_amax = jnp.max(jnp.abs(acc))

    amax_send[...] = jnp.zeros((8, 128), jnp.float32) + my_amax
    am_rdmas = []
    for k in range(1, N_DEV):
        d = (my + k) % N_DEV
        slot = N_DEV - 1 - k
        rdma = pltpu.make_async_remote_copy(
            src_ref=amax_send,
            dst_ref=amax_ref.at[slot],
            send_sem=am_send_sems.at[slot],
            recv_sem=am_recv_sems.at[slot],
            device_id=(d,),
            device_id_type=pl.DeviceIdType.MESH,
        )
        rdma.start()
        am_rdmas.append(rdma)
    for rdma in am_rdmas:
        rdma.wait()
    amax = jnp.maximum(jnp.max(amax_ref[...]), my_amax)
    scale = amax / 448.0

    q = (acc / scale).astype(jnp.float8_e4m3fn)
    q_send[...] = q
    out_ref[chunk_rows(my), :] = (q.astype(jnp.float32) * scale).astype(jnp.bfloat16)

    ag_rdmas = []
    for k in range(1, N_DEV):
        d = (my + k) % N_DEV
        slot = N_DEV - 1 - k
        rdma = pltpu.make_async_remote_copy(
            src_ref=q_send,
            dst_ref=agq_ref.at[slot],
            send_sem=ag_send_sems.at[slot],
            recv_sem=ag_recv_sems.at[slot],
            device_id=(d,),
            device_id_type=pl.DeviceIdType.MESH,
        )
        rdma.start()
        ag_rdmas.append(rdma)
    for k, rdma in enumerate(ag_rdmas, start=1):
        rdma.wait()
        c = (my - k) % N_DEV
        out_ref[chunk_rows(c), :] = (
            agq_ref[N_DEV - 1 - k].astype(jnp.float32) * scale
        ).astype(jnp.bfloat16)


def _all_reduce_quant(partial):
    n_sl = N_DEV - 1
    return pl.pallas_call(
        _ar_body,
        out_shape=jax.ShapeDtypeStruct((M, N), jnp.bfloat16),
        in_specs=[pl.BlockSpec(memory_space=pltpu.VMEM)],
        out_specs=pl.BlockSpec(memory_space=pltpu.VMEM),
        scratch_shapes=[
            pltpu.VMEM((n_sl, CHUNK, N), jnp.bfloat16),
            pltpu.VMEM((n_sl, CHUNK, N), jnp.float8_e4m3fn),
            pltpu.VMEM((CHUNK, N), jnp.float8_e4m3fn),
            pltpu.VMEM((8, 128), jnp.float32),
            pltpu.VMEM((n_sl, 8, 128), jnp.float32),
            pltpu.SemaphoreType.DMA((n_sl,)),
            pltpu.SemaphoreType.DMA((n_sl,)),
            pltpu.SemaphoreType.DMA((n_sl,)),
            pltpu.SemaphoreType.DMA((n_sl,)),
            pltpu.SemaphoreType.DMA((n_sl,)),
            pltpu.SemaphoreType.DMA((n_sl,)),
        ],
        compiler_params=pltpu.CompilerParams(
            collective_id=0,
            vmem_limit_bytes=128 * 1024 * 1024,
        ),
    )(partial)


def kernel(x, w_mat):
    partial = jnp.dot(x, w_mat, preferred_element_type=jnp.bfloat16)
    return _all_reduce_quant(partial)


# device time: 244121 ns/iter; 1.5091x vs baseline; 1.0391x over previous
import jax
import jax.numpy as jnp
from jax import lax
from jax.experimental import pallas as pl
from jax.experimental.pallas import tpu as pltpu

N_DEV = 8
M, N = 4096, 2048
CHUNK = M // N_DEV


def _ar_body(x_ref, w_ref, out_ref, send_ref, a2a_ref, agq_ref, q_send,
             amax_send, amax_ref,
             a2a_send_sems, a2a_recv_sems, ag_send_sems, ag_recv_sems,
             am_send_sems, am_recv_sems):
    my = lax.axis_index("i")

    def chunk_rows(c):
        return pl.ds(c * CHUNK, CHUNK)

    barrier_sem = pltpu.get_barrier_semaphore()
    for k in range(1, N_DEV):
        pl.semaphore_signal(
            barrier_sem, inc=1,
            device_id=((my + k) % N_DEV,),
            device_id_type=pl.DeviceIdType.MESH,
        )
    pl.semaphore_wait(barrier_sem, N_DEV - 1)

    rs_rdmas = []
    for k in range(1, N_DEV):
        d = (my + k) % N_DEV
        slot = N_DEV - 1 - k
        pj = jnp.dot(x_ref[chunk_rows(d), :], w_ref[...],
                     preferred_element_type=jnp.float32)
        send_ref[slot] = pj.astype(jnp.bfloat16)
        rdma = pltpu.make_async_remote_copy(
            src_ref=send_ref.at[slot],
            dst_ref=a2a_ref.at[slot],
            send_sem=a2a_send_sems.at[slot],
            recv_sem=a2a_recv_sems.at[slot],
            device_id=(d,),
            device_id_type=pl.DeviceIdType.MESH,
        )
        rdma.start()
        rs_rdmas.append(rdma)
    acc = jnp.dot(x_ref[chunk_rows(my), :], w_ref[...],
                  preferred_element_type=jnp.float32)
    for k, rdma in enumerate(rs_rdmas, start=1):
        rdma.wait()
        acc = acc + a2a_ref[N_DEV - 1 - k].astype(jnp.float32)
    my_amax = jnp.max(jnp.abs(acc))

    amax_send[...] = jnp.zeros((8, 128), jnp.float32) + my_amax
    am_rdmas = []
    for k in range(1, N_DEV):
        d = (my + k) % N_DEV
        slot = N_DEV - 1 - k
        rdma = pltpu.make_async_remote_copy(
            src_ref=amax_send,
            dst_ref=amax_ref.at[slot],
            send_sem=am_send_sems.at[slot],
            recv_sem=am_recv_sems.at[slot],
            device_id=(d,),
            device_id_type=pl.DeviceIdType.MESH,
        )
        rdma.start()
        am_rdmas.append(rdma)
    for rdma in am_rdmas:
        rdma.wait()
    amax = jnp.maximum(jnp.max(amax_ref[...]), my_amax)
    scale = amax / 448.0

    q = (acc / scale).astype(jnp.float8_e4m3fn)
    q_send[...] = q
    out_ref[chunk_rows(my), :] = (q.astype(jnp.float32) * scale).astype(jnp.bfloat16)

    ag_rdmas = []
    for k in range(1, N_DEV):
        d = (my + k) % N_DEV
        slot = N_DEV - 1 - k
        rdma = pltpu.make_async_remote_copy(
            src_ref=q_send,
            dst_ref=agq_ref.at[slot],
            send_sem=ag_send_sems.at[slot],
            recv_sem=ag_recv_sems.at[slot],
            device_id=(d,),
            device_id_type=pl.DeviceIdType.MESH,
        )
        rdma.start()
        ag_rdmas.append(rdma)
    for k, rdma in enumerate(ag_rdmas, start=1):
        rdma.wait()
        c = (my - k) % N_DEV
        out_ref[chunk_rows(c), :] = (
            agq_ref[N_DEV - 1 - k].astype(jnp.float32) * scale
        ).astype(jnp.bfloat16)


def _gemm_ar_quant(x, w):
    n_sl = N_DEV - 1
    return pl.pallas_call(
        _ar_body,
        out_shape=jax.ShapeDtypeStruct((M, N), jnp.bfloat16),
        in_specs=[pl.BlockSpec(memory_space=pltpu.VMEM),
                  pl.BlockSpec(memory_space=pltpu.VMEM)],
        out_specs=pl.BlockSpec(memory_space=pltpu.VMEM),
        scratch_shapes=[
            pltpu.VMEM((n_sl, CHUNK, N), jnp.bfloat16),
            pltpu.VMEM((n_sl, CHUNK, N), jnp.bfloat16),
            pltpu.VMEM((n_sl, CHUNK, N), jnp.float8_e4m3fn),
            pltpu.VMEM((CHUNK, N), jnp.float8_e4m3fn),
            pltpu.VMEM((8, 128), jnp.float32),
            pltpu.VMEM((n_sl, 8, 128), jnp.float32),
            pltpu.SemaphoreType.DMA((n_sl,)),
            pltpu.SemaphoreType.DMA((n_sl,)),
            pltpu.SemaphoreType.DMA((n_sl,)),
            pltpu.SemaphoreType.DMA((n_sl,)),
            pltpu.SemaphoreType.DMA((n_sl,)),
            pltpu.SemaphoreType.DMA((n_sl,)),
        ],
        compiler_params=pltpu.CompilerParams(
            collective_id=0,
            vmem_limit_bytes=128 * 1024 * 1024,
        ),
    )(x, w)


def kernel(x, w_mat):
    return _gemm_ar_quant(x.astype(jnp.bfloat16), w_mat.astype(jnp.bfloat16))


# device time: 151862 ns/iter; 2.4259x vs baseline; 1.6075x over previous
import jax
import jax.numpy as jnp
from jax import lax
from jax.experimental import pallas as pl
from jax.experimental.pallas import tpu as pltpu

N_DEV = 8
M, N, K = 4096, 2048, 512
CHUNK = M // N_DEV

COL_OFF = (0, 640, 1280)
COL_W = (640, 640, 768)
MASKS = ((1, 3, 4), (3, 4, 1), (4, 1, 3))


def _coset_offsets(masks):
    offs = [0]
    for m in masks:
        offs = offs + [o ^ m for o in offs]
    return offs


def _ar_body(x_ref, w_ref, out_ref,
             vals0, vals1, vals2, rbuf0, rbuf1, rbuf2, q_ref,
             amax_send, amax_ref,
             rs_send_sems, rs_recv_sems, ag_send_sems, ag_recv_sems,
             am_send_sems, am_recv_sems):
    my = lax.axis_index("i")
    vals = (vals0, vals1, vals2)
    rbuf = (rbuf0, rbuf1, rbuf2)

    def rows(idx):
        return pl.ds(idx * CHUNK, CHUNK)

    barrier_sem = pltpu.get_barrier_semaphore()
    for k in range(1, N_DEV):
        pl.semaphore_signal(
            barrier_sem, inc=1,
            device_id=((my + k) % N_DEV,),
            device_id_type=pl.DeviceIdType.MESH,
        )
    pl.semaphore_wait(barrier_sem, N_DEV - 1)

    for r in range(N_DEV):
        pj = jnp.dot(x_ref[rows(r), :], w_ref[...],
                     preferred_element_type=jnp.float32).astype(jnp.bfloat16)
        for c in range(3):
            vals[c][rows(r), :] = pj[:, COL_OFF[c]:COL_OFF[c] + COL_W[c]]

    acc = [None, None, None]
    for s in range(3):
        rdmas = []
        for c in range(3):
            m_s = MASKS[c][s]
            offs = _coset_offsets(MASKS[c][s + 1:])
            partner = my ^ m_s
            for i, o in enumerate(offs):
                rdma = pltpu.make_async_remote_copy(
                    src_ref=vals[c].at[rows(partner ^ o)],
                    dst_ref=rbuf[c].at[i],
                    send_sem=rs_send_sems.at[c, s, i],
                    recv_sem=rs_recv_sems.at[c, s, i],
                    device_id=(partner,),
                    device_id_type=pl.DeviceIdType.MESH,
                )
                rdma.start()
                rdmas.append((c, i, offs, rdma))
        for c, i, offs, rdma in rdmas:
            rdma.wait()
            o = offs[i]
            if s < 2:
                vals[c][rows(my ^ o), :] = (
                    vals[c][rows(my ^ o), :].astype(jnp.float32)
                    + rbuf[c][i].astype(jnp.float32)
                ).astype(jnp.bfloat16)
            else:
                acc[c] = (vals[c][rows(my), :].astype(jnp.float32)
                          + rbuf[c][i].astype(jnp.float32))

    my_amax = jnp.maximum(
        jnp.maximum(jnp.max(jnp.abs(acc[0])), jnp.max(jnp.abs(acc[1]))),
        jnp.max(jnp.abs(acc[2])),
    )
    amax_send[...] = jnp.zeros((8, 128), jnp.float32) + my_amax
    am_rdmas = []
    for k in range(1, N_DEV):
        d = (my + k) % N_DEV
        slot = N_DEV - 1 - k
        rdma = pltpu.make_async_remote_copy(
            src_ref=amax_send,
            dst_ref=amax_ref.at[slot],
            send_sem=am_send_sems.at[slot],
            recv_sem=am_recv_sems.at[slot],
            device_id=(d,),
            device_id_type=pl.DeviceIdType.MESH,
        )
        rdma.start()
        am_rdmas.append(rdma)
    for rdma in am_rdmas:
        rdma.wait()
    amax = jnp.maximum(jnp.max(amax_ref[...]), my_amax)
    scale = amax / 448.0

    for c in range(3):
        q_ref[rows(my), pl.ds(COL_OFF[c], COL_W[c])] = (
            acc[c] / scale).astype(jnp.float8_e4m3fn)

    for g in range(3):
        rdmas = []
        for c in range(3):
            m_g = MASKS[c][2 - g]
            offs = _coset_offsets(MASKS[c][3 - g:])
            partner = my ^ m_g
            for i, o in enumerate(offs):
                rdma = pltpu.make_async_remote_copy(
                    src_ref=q_ref.at[rows(my ^ o), pl.ds(COL_OFF[c], COL_W[c])],
                    dst_ref=q_ref.at[rows(my ^ o), pl.ds(COL_OFF[c], COL_W[c])],
                    send_sem=ag_send_sems.at[c, g, i],
                    recv_sem=ag_recv_sems.at[c, g, i],
                    device_id=(partner,),
                    device_id_type=pl.DeviceIdType.MESH,
                )
                rdma.start()
                rdmas.append(rdma)
        for rdma in rdmas:
            rdma.wait()

    out_ref[...] = (q_ref[...].astype(jnp.float32) * scale).astype(jnp.bfloat16)


def _gemm_ar_quant(x, w):
    n_sl = N_DEV - 1
    return pl.pallas_call(
        _ar_body,
        out_shape=jax.ShapeDtypeStruct((M, N), jnp.bfloat16),
        in_specs=[pl.BlockSpec(memory_space=pltpu.VMEM),
                  pl.BlockSpec(memory_space=pltpu.VMEM)],
        out_specs=pl.BlockSpec(memory_space=pltpu.VMEM),
        scratch_shapes=[
            pltpu.VMEM((M, COL_W[0]), jnp.bfloat16),
            pltpu.VMEM((M, COL_W[1]), jnp.bfloat16),
            pltpu.VMEM((M, COL_W[2]), jnp.bfloat16),
            pltpu.VMEM((4, CHUNK, COL_W[0]), jnp.bfloat16),
            pltpu.VMEM((4, CHUNK, COL_W[1]), jnp.bfloat16),
            pltpu.VMEM((4, CHUNK, COL_W[2]), jnp.bfloat16),
            pltpu.VMEM((M, N), jnp.float8_e4m3fn),
            pltpu.VMEM((8, 128), jnp.float32),
            pltpu.VMEM((n_sl, 8, 128), jnp.float32),
            pltpu.SemaphoreType.DMA((3, 3, 4)),
            pltpu.SemaphoreType.DMA((3, 3, 4)),
            pltpu.SemaphoreType.DMA((3, 3, 4)),
            pltpu.SemaphoreType.DMA((3, 3, 4)),
            pltpu.SemaphoreType.DMA((n_sl,)),
            pltpu.SemaphoreType.DMA((n_sl,)),
        ],
        compiler_params=pltpu.CompilerParams(
            collective_id=0,
            vmem_limit_bytes=128 * 1024 * 1024,
        ),
    )(x, w)


def kernel(x, w_mat):
    return _gemm_ar_quant(x.astype(jnp.bfloat16), w_mat.astype(jnp.bfloat16))
